# Initial kernel scaffold; baseline (speedup 1.0000x reference)
import functools

import jax
import jax.numpy as jnp
from jax import lax
from jax.experimental import pallas as pl
from jax.experimental.pallas import tpu as pltpu
from jax.experimental.pallas import tpu_sc as plsc

BATCH = 4096
SEQ = 200
N_TOKENS = 10
EMBED_DIM = 64
N_GATHER = SEQ - N_TOKENS
CHUNK = 96
N_CHUNKS = 2

_SC_INFO = plsc.get_sparse_core_info()
NUM_WORKERS = _SC_INFO.num_cores * _SC_INFO.num_subcores
ROWS_PER_WORKER = BATCH // NUM_WORKERS


@functools.partial(
    pl.kernel,
    out_type=jax.ShapeDtypeStruct((BATCH, SEQ, EMBED_DIM), jnp.float32),
    mesh=plsc.VectorSubcoreMesh(core_axis_name="c", subcore_axis_name="s"),
    scratch_types=[
        pltpu.VMEM((N_CHUNKS, CHUNK), jnp.int32),
        pltpu.VMEM((N_TOKENS + N_CHUNKS * CHUNK, EMBED_DIM), jnp.float32),
        pltpu.SemaphoreType.DMA,
    ],
)
def _soft_embedding_sc(idx_hbm, table_hbm, learned_hbm, out_hbm,
                       idx_v, buf_v, sem):
    wid = lax.axis_index("s") * _SC_INFO.num_cores + lax.axis_index("c")
    base = wid * ROWS_PER_WORKER

    pltpu.sync_copy(learned_hbm, buf_v.at[pl.ds(0, N_TOKENS)])

    @pl.loop(0, ROWS_PER_WORKER)
    def _(i):
        b = base + i
        pltpu.sync_copy(idx_hbm.at[b], idx_v)
        cp0 = pltpu.async_copy(
            table_hbm.at[idx_v.at[0]],
            buf_v.at[pl.ds(N_TOKENS, CHUNK)], sem)
        cp1 = pltpu.async_copy(
            table_hbm.at[idx_v.at[1]],
            buf_v.at[pl.ds(N_TOKENS + CHUNK, CHUNK)], sem)
        cp0.wait()
        cp1.wait()
        pltpu.sync_copy(buf_v.at[pl.ds(0, SEQ)], out_hbm.at[b])


def kernel(tokens, wte_weight, learned_embedding):
    idx = tokens[:, N_TOKENS:].astype(jnp.int32)
    idx = jnp.pad(idx, ((0, 0), (0, N_CHUNKS * CHUNK - N_GATHER)))
    idx = idx.reshape(BATCH, N_CHUNKS, CHUNK)
    return _soft_embedding_sc(idx, wte_weight, learned_embedding)


# speedup vs baseline: 1.2012x; 1.2012x over previous
import functools

import jax
import jax.numpy as jnp
from jax import lax
from jax.experimental import pallas as pl
from jax.experimental.pallas import tpu as pltpu
from jax.experimental.pallas import tpu_sc as plsc

BATCH = 4096
SEQ = 200
N_TOKENS = 10
EMBED_DIM = 64
N_GATHER = SEQ - N_TOKENS
CHUNK = 96
N_CHUNKS = 2

_SC_INFO = plsc.get_sparse_core_info()
NUM_WORKERS = _SC_INFO.num_cores * _SC_INFO.num_subcores
ROWS_PER_WORKER = BATCH // NUM_WORKERS


@functools.partial(
    pl.kernel,
    out_type=jax.ShapeDtypeStruct((BATCH, SEQ, EMBED_DIM), jnp.float32),
    mesh=plsc.VectorSubcoreMesh(core_axis_name="c", subcore_axis_name="s"),
    scratch_types=[
        pltpu.VMEM((N_CHUNKS, CHUNK), jnp.int32),
        pltpu.VMEM((N_TOKENS + N_CHUNKS * CHUNK, EMBED_DIM), jnp.float32),
        pltpu.SemaphoreType.DMA,
    ],
    compiler_params=pltpu.CompilerParams(use_tc_tiling_on_sc=False),
)
def _soft_embedding_sc(idx_hbm, table_hbm, learned_hbm, out_hbm,
                       idx_v, buf_v, sem):
    wid = lax.axis_index("s") * _SC_INFO.num_cores + lax.axis_index("c")
    base = wid * ROWS_PER_WORKER

    pltpu.sync_copy(learned_hbm, buf_v.at[pl.ds(0, N_TOKENS)])

    @pl.loop(0, ROWS_PER_WORKER)
    def _(i):
        b = base + i
        pltpu.sync_copy(idx_hbm.at[b], idx_v)
        cp0 = pltpu.async_copy(
            table_hbm.at[idx_v.at[0]],
            buf_v.at[pl.ds(N_TOKENS, CHUNK)], sem)
        cp1 = pltpu.async_copy(
            table_hbm.at[idx_v.at[1]],
            buf_v.at[pl.ds(N_TOKENS + CHUNK, CHUNK)], sem)
        cp0.wait()
        cp1.wait()
        pltpu.sync_copy(buf_v.at[pl.ds(0, SEQ)], out_hbm.at[b])


def kernel(tokens, wte_weight, learned_embedding):
    idx = tokens[:, N_TOKENS:].astype(jnp.int32)
    idx = jnp.pad(idx, ((0, 0), (0, N_CHUNKS * CHUNK - N_GATHER)))
    idx = idx.reshape(BATCH, N_CHUNKS, CHUNK)
    return _soft_embedding_sc(idx, wte_weight, learned_embedding)


# speedup vs baseline: 1.4911x; 1.2413x over previous
import functools

import jax
import jax.numpy as jnp
from jax import lax
from jax.experimental import pallas as pl
from jax.experimental.pallas import tpu as pltpu
from jax.experimental.pallas import tpu_sc as plsc

BATCH = 4096
SEQ = 200
N_TOKENS = 10
EMBED_DIM = 64
CHUNK = 96
CHUNK1_START = SEQ - CHUNK

_SC_INFO = plsc.get_sparse_core_info()
NUM_WORKERS = _SC_INFO.num_cores * _SC_INFO.num_subcores
ROWS_PER_WORKER = BATCH // NUM_WORKERS
R = 4
G = ROWS_PER_WORKER // R
NBUF = 2


@functools.partial(
    pl.kernel,
    out_type=jax.ShapeDtypeStruct((BATCH, SEQ, EMBED_DIM), jnp.float32),
    mesh=plsc.VectorSubcoreMesh(core_axis_name="c", subcore_axis_name="s"),
    scratch_types=[
        pltpu.VMEM((NBUF, R, 2, CHUNK), jnp.int32),
        pltpu.VMEM((NBUF, R, SEQ, EMBED_DIM), jnp.float32),
        [pltpu.SemaphoreType.DMA] * NBUF,
        [pltpu.SemaphoreType.DMA] * NBUF,
        [pltpu.SemaphoreType.DMA] * NBUF,
    ],
    compiler_params=pltpu.CompilerParams(use_tc_tiling_on_sc=False),
)
def _soft_embedding_sc(idx_hbm, table_hbm, learned_hbm, out_hbm,
                       idx_v, buf_v, sem_idx, sem_g, sem_out):
    wid = lax.axis_index("s") * _SC_INFO.num_cores + lax.axis_index("c")
    base = wid * ROWS_PER_WORKER

    for n in range(NBUF):
        for r in range(R):
            pltpu.sync_copy(learned_hbm, buf_v.at[n, r, pl.ds(0, N_TOKENS)])

    pltpu.async_copy(idx_hbm.at[pl.ds(base, R)], idx_v.at[0], sem_idx[0])

    @pl.loop(0, G, step=NBUF)
    def _(g):
        for n in range(NBUF):
            gi = g + n
            nb = (n + 1) % NBUF

            @pl.when(gi + 1 < G)
            def _():
                pltpu.async_copy(
                    idx_hbm.at[pl.ds(base + (gi + 1) * R, R)],
                    idx_v.at[nb], sem_idx[nb])

            pltpu.make_async_copy(
                idx_hbm.at[pl.ds(base, R)], idx_v.at[n], sem_idx[n]).wait()

            @pl.when(gi >= NBUF)
            def _():
                pltpu.make_async_copy(
                    buf_v.at[n], out_hbm.at[pl.ds(base, R)],
                    sem_out[n]).wait()

            for r in range(R):
                pltpu.async_copy(
                    table_hbm.at[idx_v.at[n, r, 0]],
                    buf_v.at[n, r, pl.ds(N_TOKENS, CHUNK)], sem_g[n])
                pltpu.async_copy(
                    table_hbm.at[idx_v.at[n, r, 1]],
                    buf_v.at[n, r, pl.ds(CHUNK1_START, CHUNK)], sem_g[n])
            for r in range(R):
                pltpu.make_async_copy(
                    table_hbm.at[idx_v.at[n, r, 0]],
                    buf_v.at[n, r, pl.ds(N_TOKENS, CHUNK)], sem_g[n]).wait()
                pltpu.make_async_copy(
                    table_hbm.at[idx_v.at[n, r, 1]],
                    buf_v.at[n, r, pl.ds(CHUNK1_START, CHUNK)],
                    sem_g[n]).wait()

            pltpu.async_copy(
                buf_v.at[n], out_hbm.at[pl.ds(base + gi * R, R)], sem_out[n])

    for n in range(NBUF):
        pltpu.make_async_copy(
            buf_v.at[n], out_hbm.at[pl.ds(base, R)], sem_out[n]).wait()


def kernel(tokens, wte_weight, learned_embedding):
    tok32 = tokens.astype(jnp.int32)
    idx = jnp.stack(
        [tok32[:, N_TOKENS:N_TOKENS + CHUNK],
         tok32[:, CHUNK1_START:SEQ]], axis=1)
    return _soft_embedding_sc(idx, wte_weight, learned_embedding)


# speedup vs baseline: 1.6170x; 1.0844x over previous
import functools

import jax
import jax.numpy as jnp
from jax import lax
from jax.experimental import pallas as pl
from jax.experimental.pallas import tpu as pltpu
from jax.experimental.pallas import tpu_sc as plsc

BATCH = 4096
SEQ = 200
N_TOKENS = 10
EMBED_DIM = 64
CHUNK = 96
CHUNK1_START = SEQ - CHUNK
IDX_PER_ROW = 2 * CHUNK

_SC_INFO = plsc.get_sparse_core_info()
NUM_WORKERS = _SC_INFO.num_cores * _SC_INFO.num_subcores
ROWS_PER_WORKER = BATCH // NUM_WORKERS
R = 4
G = ROWS_PER_WORKER // R
NBUF = 2


@functools.partial(
    pl.kernel,
    out_type=jax.ShapeDtypeStruct((BATCH * SEQ, EMBED_DIM), jnp.float32),
    mesh=plsc.VectorSubcoreMesh(core_axis_name="c", subcore_axis_name="s"),
    scratch_types=[
        pltpu.VMEM((NBUF, R * IDX_PER_ROW), jnp.int32),
        pltpu.VMEM((NBUF, R * SEQ, EMBED_DIM), jnp.float32),
        [pltpu.SemaphoreType.DMA] * NBUF,
        [pltpu.SemaphoreType.DMA] * NBUF,
        [pltpu.SemaphoreType.DMA] * NBUF,
    ],
    compiler_params=pltpu.CompilerParams(use_tc_tiling_on_sc=False),
)
def _soft_embedding_sc(idx_hbm, table_hbm, learned_hbm, out_hbm,
                       idx_v, buf_v, sem_idx, sem_g, sem_out):
    wid = lax.axis_index("s") * _SC_INFO.num_cores + lax.axis_index("c")
    base = wid * ROWS_PER_WORKER

    for n in range(NBUF):
        for r in range(R):
            pltpu.sync_copy(learned_hbm,
                            buf_v.at[n, pl.ds(r * SEQ, N_TOKENS)])

    pltpu.async_copy(idx_hbm.at[pl.ds(base * IDX_PER_ROW, R * IDX_PER_ROW)],
                     idx_v.at[0], sem_idx[0])

    @pl.loop(0, G, step=NBUF)
    def _(g):
        for n in range(NBUF):
            gi = g + n
            nb = (n + 1) % NBUF

            @pl.when(gi + 1 < G)
            def _():
                pltpu.async_copy(
                    idx_hbm.at[pl.ds((base + (gi + 1) * R) * IDX_PER_ROW,
                                     R * IDX_PER_ROW)],
                    idx_v.at[nb], sem_idx[nb])

            pltpu.make_async_copy(
                idx_hbm.at[pl.ds(0, R * IDX_PER_ROW)],
                idx_v.at[n], sem_idx[n]).wait()

            @pl.when(gi >= NBUF)
            def _():
                pltpu.make_async_copy(
                    buf_v.at[n], out_hbm.at[pl.ds(0, R * SEQ)],
                    sem_out[n]).wait()

            for r in range(R):
                pltpu.async_copy(
                    table_hbm.at[idx_v.at[n, pl.ds(r * IDX_PER_ROW, CHUNK)]],
                    buf_v.at[n, pl.ds(r * SEQ + N_TOKENS, CHUNK)], sem_g[n])
                pltpu.async_copy(
                    table_hbm.at[idx_v.at[n, pl.ds(r * IDX_PER_ROW + CHUNK, CHUNK)]],
                    buf_v.at[n, pl.ds(r * SEQ + CHUNK1_START, CHUNK)],
                    sem_g[n])
            for r in range(R):
                pltpu.make_async_copy(
                    table_hbm.at[idx_v.at[n, pl.ds(r * IDX_PER_ROW, CHUNK)]],
                    buf_v.at[n, pl.ds(r * SEQ + N_TOKENS, CHUNK)],
                    sem_g[n]).wait()
                pltpu.make_async_copy(
                    table_hbm.at[idx_v.at[n, pl.ds(r * IDX_PER_ROW + CHUNK, CHUNK)]],
                    buf_v.at[n, pl.ds(r * SEQ + CHUNK1_START, CHUNK)],
                    sem_g[n]).wait()

            pltpu.async_copy(
                buf_v.at[n],
                out_hbm.at[pl.ds((base + gi * R) * SEQ, R * SEQ)],
                sem_out[n])

    for n in range(NBUF):
        pltpu.make_async_copy(
            buf_v.at[n], out_hbm.at[pl.ds(0, R * SEQ)], sem_out[n]).wait()


def kernel(tokens, wte_weight, learned_embedding):
    tok32 = tokens.astype(jnp.int32)
    idx = jnp.stack(
        [tok32[:, N_TOKENS:N_TOKENS + CHUNK],
         tok32[:, CHUNK1_START:SEQ]], axis=1)
    idx = idx.reshape(BATCH * IDX_PER_ROW)
    wte_flat = lax.optimization_barrier(wte_weight.reshape(-1))
    wte_rows = wte_flat.reshape(wte_weight.shape)
    out = _soft_embedding_sc(idx, wte_rows, learned_embedding)
    return out.reshape(BATCH, SEQ, EMBED_DIM)
